# Initial kernel scaffold; baseline (speedup 1.0000x reference)
import jax
import jax.numpy as jnp
from jax.experimental import pallas as pl


def kernel(x, edge_index, adj, pos_edge, params):
    raise NotImplementedError("write your pallas kernel here")


# speedup vs baseline: 2.0341x; 2.0341x over previous
import functools

import jax
import jax.numpy as jnp
import numpy as np
from jax import lax
from jax.experimental import pallas as pl
from jax.experimental.pallas import tpu as pltpu
from jax.experimental.pallas import tpu_sc as plsc

N = 10000
D = 128
H = 256
E = 320000
Q = 4096

NC = 2
NS = 16
KCH = 80
RT_A = 632
RT_LAST = N - (NS - 1) * RT_A

_BN_INV = 1.0 / np.sqrt(1.0 + 1e-5)


def _sc_mesh():
    return plsc.VectorSubcoreMesh(core_axis_name="c", subcore_axis_name="s",
                                  num_cores=NC, num_subcores=NS)


def _make_sc_agg(feature_split):
    ept = (E // NS) if feature_split else (E // NC // NS)
    nch = ept // KCH

    @functools.partial(
        pl.kernel,
        out_type=jax.ShapeDtypeStruct((2 * N, D), jnp.float32),
        mesh=_sc_mesh(),
        scratch_types=[
            pltpu.VMEM((KCH,), jnp.int32),
            pltpu.VMEM((KCH,), jnp.int32),
            pltpu.VMEM((KCH, D), jnp.float32),
            pltpu.VMEM_SHARED((N, D), jnp.float32),
            pltpu.SemaphoreType.DMA,
        ],
    )
    def k(rows_hbm, src_hbm, dst_hbm, zeros_hbm, out_hbm,
          src_v, dst_v, rows_v, acc, sem):
        c = lax.axis_index("c")
        s = lax.axis_index("s")
        r0 = s * RT_A

        @pl.when(s < NS - 1)
        def _():
            pltpu.sync_copy(zeros_hbm, acc.at[pl.ds(r0, RT_A)])

        @pl.when(s == NS - 1)
        def _():
            pltpu.sync_copy(zeros_hbm.at[pl.ds(0, RT_LAST)],
                            acc.at[pl.ds((NS - 1) * RT_A, RT_LAST)])

        plsc.subcore_barrier()
        if feature_split:
            ebase = s * ept
        else:
            ebase = c * (E // NC) + s * ept

        def body(i, carry):
            off = ebase + i * KCH
            pltpu.sync_copy(src_hbm.at[pl.ds(off, KCH)], src_v)
            pltpu.sync_copy(dst_hbm.at[pl.ds(off, KCH)], dst_v)
            if feature_split:
                for j in range(KCH // 16):
                    sl = pl.ds(j * 16, 16)
                    src_v[sl] = src_v[sl] + c * N
            pltpu.async_copy(rows_hbm.at[src_v], rows_v, sem).wait()
            pltpu.sync_copy(rows_v, acc.at[dst_v], add=True)
            return carry

        lax.fori_loop(0, nch, body, 0)
        plsc.subcore_barrier()

        @pl.when(s < NS - 1)
        def _():
            pltpu.sync_copy(acc.at[pl.ds(r0, RT_A)],
                            out_hbm.at[pl.ds(c * N + r0, RT_A)])

        @pl.when(s == NS - 1)
        def _():
            pltpu.sync_copy(acc.at[pl.ds((NS - 1) * RT_A, RT_LAST)],
                            out_hbm.at[pl.ds(c * N + (NS - 1) * RT_A, RT_LAST)])

    return k


_sc_agg_edges = _make_sc_agg(False)
_sc_agg_feat = _make_sc_agg(True)

QPT = Q // (NC * NS)
_CN_GRP = 4
_XCH = 16
NP_ = 10240


def _make_sc_cn():
    qpt = Q // (NC * NS)

    @functools.partial(
        pl.kernel,
        out_type=[
            jax.ShapeDtypeStruct((Q * NP_,), jnp.float32),
            jax.ShapeDtypeStruct((Q, H), jnp.float32),
            jax.ShapeDtypeStruct((Q, H), jnp.float32),
        ],
        mesh=_sc_mesh(),
        scratch_types=[
            pltpu.VMEM((_XCH,), jnp.int32),
            pltpu.VMEM((_XCH, H), jnp.float32),
            pltpu.VMEM((8,), jnp.int32),
            pltpu.VMEM((8, NP_), jnp.float32),
            pltpu.VMEM((4 * NP_,), jnp.float32),
            pltpu.SemaphoreType.DMA,
        ],
    )
    def _sc_cn(adj_hbm, pairs_hbm, posf_hbm, xl_hbm,
               cn_hbm, xi_hbm, xj_hbm,
               idxx_v, xrows_v, idx8_v, rows_v, cnbuf_v, sem):
        c = lax.axis_index("c")
        s = lax.axis_index("s")
        wid = s * NC + c
        q0 = wid * qpt

        for t in range(qpt // _XCH):
            qt = q0 + t * _XCH
            pltpu.sync_copy(posf_hbm.at[pl.ds(qt, _XCH)], idxx_v)
            pltpu.async_copy(xl_hbm.at[idxx_v], xrows_v, sem).wait()
            pltpu.sync_copy(xrows_v, xi_hbm.at[pl.ds(qt, _XCH)])
            pltpu.sync_copy(posf_hbm.at[pl.ds(Q + qt, _XCH)], idxx_v)
            pltpu.async_copy(xl_hbm.at[idxx_v], xrows_v, sem).wait()
            pltpu.sync_copy(xrows_v, xj_hbm.at[pl.ds(qt, _XCH)])

        def group(t2, carry):
            qb = q0 + t2 * 4
            pltpu.sync_copy(pairs_hbm.at[pl.ds(2 * qb, 8)], idx8_v)
            pltpu.async_copy(adj_hbm.at[idx8_v], rows_v, sem).wait()
            for u in range(4):

                def cols(cc, c2, u=u):
                    for g in range(5):
                        co = (cc * 5 + g) * 16
                        sl = pl.ds(co, 16)
                        cnbuf_v[pl.ds(u * NP_ + co, 16)] = (
                            rows_v[2 * u, sl] * rows_v[2 * u + 1, sl])
                    return c2

                lax.fori_loop(0, NP_ // 80, cols, 0)
            for u in range(4):
                pltpu.sync_copy(cnbuf_v.at[pl.ds(u * NP_, NP_)],
                                cn_hbm.at[pl.ds((qb + u) * NP_, NP_)])
            return carry

        lax.fori_loop(0, qpt // 4, group, 0)

    return _sc_cn


_sc_cn = _make_sc_cn()


def _ln(h, g, b):
    m = jnp.mean(h, axis=-1, keepdims=True)
    v = jnp.mean((h - m) ** 2, axis=-1, keepdims=True)
    return (h - m) * jax.lax.rsqrt(v + 1e-5) * g + b


BN_ = 1000


def _tc_mlp1_body(x_ref, aggp_ref, w1a_ref, b1a_ref, w1b_ref, b1b_ref,
                  g_ref, bb_ref, eps_ref, out_ref):
    h = x_ref[...] * (1.0 + eps_ref[0, 0]) + aggp_ref[0] + aggp_ref[1]
    h = jnp.maximum(jnp.dot(h, w1a_ref[...],
                            preferred_element_type=jnp.float32) + b1a_ref[...], 0.0)
    h = jnp.maximum(jnp.dot(h, w1b_ref[...],
                            preferred_element_type=jnp.float32) + b1b_ref[...], 0.0)
    y = h * (_BN_INV * g_ref[...]) + bb_ref[...]
    out_ref[0] = y[:, :D]
    out_ref[1] = y[:, D:]


def _tc_mlp1(x, aggp, w1a, b1a, w1b, b1b, g, b, eps):
    grid = (N // BN_,)
    return pl.pallas_call(
        _tc_mlp1_body,
        grid=grid,
        in_specs=[
            pl.BlockSpec((BN_, D), lambda i: (i, 0)),
            pl.BlockSpec((2, BN_, D), lambda i: (0, i, 0)),
            pl.BlockSpec((D, H), lambda i: (0, 0)),
            pl.BlockSpec((1, H), lambda i: (0, 0)),
            pl.BlockSpec((H, H), lambda i: (0, 0)),
            pl.BlockSpec((1, H), lambda i: (0, 0)),
            pl.BlockSpec((1, H), lambda i: (0, 0)),
            pl.BlockSpec((1, H), lambda i: (0, 0)),
            pl.BlockSpec((1, 1), lambda i: (0, 0)),
        ],
        out_specs=pl.BlockSpec((2, BN_, D), lambda i: (0, i, 0)),
        out_shape=jax.ShapeDtypeStruct((2, N, D), jnp.float32),
        compiler_params=pltpu.CompilerParams(
            dimension_semantics=("parallel",)),
    )(x, aggp, w1a, b1a, w1b, b1b, g, b, eps)


def _tc_mlp2_body(x1h_ref, a2h_ref, w2a_ref, b2a_ref, g2_ref, bb2_ref,
                  wl1_ref, bl1_ref, wx1_ref, bx1_ref, wx2_ref, bx2_ref,
                  lng_ref, lnb_ref, eps_ref, xl_ref, x3_ref):
    e = 1.0 + eps_ref[0, 0]
    ta = x1h_ref[0] * e + a2h_ref[0]
    tb = x1h_ref[1] * e + a2h_ref[1]
    h = (jnp.dot(ta, w2a_ref[:D, :], preferred_element_type=jnp.float32)
         + jnp.dot(tb, w2a_ref[D:, :], preferred_element_type=jnp.float32)
         + b2a_ref[...])
    h = jnp.maximum(h, 0.0)
    x2 = h * (_BN_INV * g2_ref[...]) + bb2_ref[...]
    xl = jnp.dot(x2, wl1_ref[...], preferred_element_type=jnp.float32) + bl1_ref[...]
    hx = jnp.maximum(jnp.dot(xl, wx1_ref[...],
                             preferred_element_type=jnp.float32) + bx1_ref[...], 0.0)
    hx = jnp.dot(hx, wx2_ref[...], preferred_element_type=jnp.float32) + bx2_ref[...]
    hx = jnp.maximum(_ln(hx, lng_ref[...], lnb_ref[...]), 0.0)
    xl_ref[...] = xl
    x3_ref[...] = xl + hx


def _tc_mlp2(x1h, a2h, w2a, b2a, g2, b2, wl1, bl1, wx1, bx1, wx2, bx2,
             lng, lnb, eps):
    grid = (N // BN_,)
    hh = pl.BlockSpec((H, H), lambda i: (0, 0))
    vh = pl.BlockSpec((1, H), lambda i: (0, 0))
    return pl.pallas_call(
        _tc_mlp2_body,
        grid=grid,
        in_specs=[
            pl.BlockSpec((2, BN_, D), lambda i: (0, i, 0)),
            pl.BlockSpec((2, BN_, D), lambda i: (0, i, 0)),
            hh, vh, vh, vh,
            hh, vh, hh, vh, hh, vh,
            vh, vh,
            pl.BlockSpec((1, 1), lambda i: (0, 0)),
        ],
        out_specs=[
            pl.BlockSpec((BN_, H), lambda i: (i, 0)),
            pl.BlockSpec((BN_, H), lambda i: (i, 0)),
        ],
        out_shape=[
            jax.ShapeDtypeStruct((N, H), jnp.float32),
            jax.ShapeDtypeStruct((N, H), jnp.float32),
        ],
        compiler_params=pltpu.CompilerParams(
            dimension_semantics=("parallel",)),
    )(x1h, a2h, w2a, b2a, g2, b2, wl1, bl1, wx1, bx1, wx2, bx2, lng, lnb, eps)


BQ = 512
BQC = 256


def _tc_cnmm_body(cn_ref, x3_ref, o_ref):
    o_ref[...] = jnp.dot(cn_ref[...], x3_ref[...],
                         preferred_element_type=jnp.float32)


def _tc_cnmm(cn, x3):
    grid = (Q // BQC,)
    return pl.pallas_call(
        _tc_cnmm_body,
        grid=grid,
        in_specs=[
            pl.BlockSpec((BQC, NP_), lambda i: (i, 0)),
            pl.BlockSpec((NP_, H), lambda i: (0, 0)),
        ],
        out_specs=pl.BlockSpec((BQC, H), lambda i: (i, 0)),
        out_shape=jax.ShapeDtypeStruct((Q, H), jnp.float32),
        compiler_params=pltpu.CompilerParams(
            dimension_semantics=("parallel",)),
    )(cn, x3)


def _tc_final_body(xcn_ref, xi_ref, xj_ref,
                   wi1_ref, bi1_ref, lnig_ref, lnib_ref, wi2_ref, bi2_ref,
                   wc1_ref, bc1_ref, wc2_ref, bc2_ref, lncg_ref, lncb_ref,
                   wc3_ref, bc3_ref, beta_ref,
                   wl1_ref, bl1_ref, ln1g_ref, ln1b_ref,
                   wl2_ref, bl2_ref, ln2g_ref, ln2b_ref,
                   wl3_ref, bl3_ref, o_ref):
    dot = lambda a, w, b: jnp.dot(a, w[...],
                                  preferred_element_type=jnp.float32) + b[...]
    hij = dot(xi_ref[...] * xj_ref[...], wi1_ref, bi1_ref)
    hij = jnp.maximum(_ln(hij, lnig_ref[...], lnib_ref[...]), 0.0)
    xij = dot(hij, wi2_ref, bi2_ref)
    hc = jnp.maximum(dot(xcn_ref[...], wc1_ref, bc1_ref), 0.0)
    hc = dot(hc, wc2_ref, bc2_ref)
    hc = jnp.maximum(_ln(hc, lncg_ref[...], lncb_ref[...]), 0.0)
    hc = dot(hc, wc3_ref, bc3_ref)
    pre = hc * beta_ref[0, 0] + xij
    o = dot(pre, wl1_ref, bl1_ref)
    o = jnp.maximum(_ln(o, ln1g_ref[...], ln1b_ref[...]), 0.0)
    o = dot(o, wl2_ref, bl2_ref)
    o = jnp.maximum(_ln(o, ln2g_ref[...], ln2b_ref[...]), 0.0)
    o_ref[...] = dot(o, wl3_ref, bl3_ref)


def _tc_final(xcn, xi, xj, args):
    grid = (Q // BQ,)
    hh = pl.BlockSpec((H, H), lambda i: (0, 0))
    vh = pl.BlockSpec((1, H), lambda i: (0, 0))
    qh = pl.BlockSpec((BQ, H), lambda i: (i, 0))
    return pl.pallas_call(
        _tc_final_body,
        grid=grid,
        in_specs=[
            qh, qh, qh,
            hh, vh, vh, vh, hh, vh,
            hh, vh, hh, vh, vh, vh, hh, vh,
            pl.BlockSpec((1, 1), lambda i: (0, 0)),
            hh, vh, vh, vh,
            hh, vh, vh, vh,
            pl.BlockSpec((H, D), lambda i: (0, 0)),
            pl.BlockSpec((1, D), lambda i: (0, 0)),
        ],
        out_specs=pl.BlockSpec((BQ, D), lambda i: (i, 0)),
        out_shape=jax.ShapeDtypeStruct((Q, D), jnp.float32),
        compiler_params=pltpu.CompilerParams(
            dimension_semantics=("parallel",)),
    )(xcn, xi, xj, *args)


def kernel(x, edge_index, adj, pos_edge, params):
    p = params
    src = edge_index[0]
    dst = edge_index[1]
    zeros = jnp.zeros((RT_A, D), jnp.float32)
    r2 = lambda v: v.reshape(1, -1)

    aggp = _sc_agg_edges(x, src, dst, zeros).reshape(2, N, D)
    x1h = _tc_mlp1(x, aggp, p['W1a'], r2(p['b1a']), p['W1b'], r2(p['b1b']),
                   r2(p['bn1_g']), r2(p['bn1_b']),
                   p['eps1'].reshape(1, 1).astype(jnp.float32))
    x1flat = x1h.reshape(2 * N, D)
    a2h = _sc_agg_feat(x1flat, src, dst, zeros).reshape(2, N, D)
    xl, x3 = _tc_mlp2(x1h, a2h, p['W2a'], r2(p['b2a']),
                      r2(p['bn2_g']), r2(p['bn2_b']),
                      p['Wl1'], r2(p['bl1']), p['Wx1'], r2(p['bx1']),
                      p['Wx2'], r2(p['bx2']), r2(p['lnx_g']), r2(p['lnx_b']),
                      p['eps2'].reshape(1, 1).astype(jnp.float32))

    pairs = jnp.stack([pos_edge[0], pos_edge[1]], axis=1).ravel()
    posf = pos_edge.ravel()
    adjp = jnp.pad(adj, ((0, 0), (0, NP_ - N)))
    x3p = jnp.pad(x3, ((0, NP_ - N), (0, 0)))
    cn, xi, xj = _sc_cn(adjp, pairs, posf, xl)
    xcn = _tc_cnmm(cn.reshape(Q, NP_), x3p)

    wl3 = jnp.pad(p['WL3'], ((0, 0), (0, D - p['WL3'].shape[1])))
    bl3 = jnp.pad(p['bL3'], (0, D - p['bL3'].shape[0])).reshape(1, D)
    args = (p['Wi1'], r2(p['bi1']), r2(p['lni_g']), r2(p['lni_b']),
            p['Wi2'], r2(p['bi2']),
            p['Wc1'], r2(p['bc1']), p['Wc2'], r2(p['bc2']),
            r2(p['lnc_g']), r2(p['lnc_b']), p['Wc3'], r2(p['bc3']),
            p['beta'].reshape(1, 1).astype(jnp.float32),
            p['WL1'], r2(p['bL1']), r2(p['lnL1_g']), r2(p['lnL1_b']),
            p['WL2'], r2(p['bL2']), r2(p['lnL2_g']), r2(p['lnL2_b']),
            wl3, bl3)
    o = _tc_final(xcn, xi, xj, args)
    return o[:, :7]
